# baseline (speedup 1.0000x reference)
import functools
import math

import jax
import jax.numpy as jnp
from jax import lax
from jax.experimental import pallas as pl
from jax.experimental.pallas import tpu as pltpu
from jax.experimental.pallas import tpu_sc as plsc

NC = 2
NS = 16
NW = NC * NS

SUB = 128
SUBS_PER_CHUNK = 4
CHUNK = SUB * SUBS_PER_CHUNK


@functools.lru_cache(maxsize=None)
def _build(B, D):
    assert B % (NW * CHUNK) == 0, (B, NW * CHUNK)
    n_per_w = B // NW
    n_chunks = n_per_w // CHUNK
    xrows_per_w = n_per_w // SUB
    scale = math.sqrt(D)

    mesh = plsc.VectorSubcoreMesh(core_axis_name="c", subcore_axis_name="s")

    @functools.partial(
        pl.kernel,
        mesh=mesh,
        compiler_params=pltpu.CompilerParams(use_tc_tiling_on_sc=False),
        out_type=jax.ShapeDtypeStruct((B, D), jnp.float32),
        scratch_types=[
            pltpu.VMEM((SUBS_PER_CHUNK, SUB), jnp.int32),
            pltpu.VMEM((CHUNK, D), jnp.float32),
            pltpu.SemaphoreType.DMA,
        ],
    )
    def k(x_hbm, table_hbm, out_hbm, idx_v, rows_v, gsem):
        wid = lax.axis_index("s") * NC + lax.axis_index("c")
        xrow0 = wid * xrows_per_w
        out0 = wid * n_per_w

        def chunk_body(g, carry):
            pltpu.sync_copy(x_hbm.at[pl.ds(xrow0 + g * SUBS_PER_CHUNK,
                                           SUBS_PER_CHUNK)], idx_v)
            cps = [
                pltpu.async_copy(table_hbm.at[idx_v.at[j]],
                                 rows_v.at[pl.ds(j * SUB, SUB)], gsem)
                for j in range(SUBS_PER_CHUNK)
            ]
            for cp in cps:
                cp.wait()

            def scale_row(r, c):
                for l in range(D // 16):
                    sl = pl.ds(l * 16, 16)
                    rows_v[r, sl] = rows_v[r, sl] * scale
                return c

            lax.fori_loop(0, CHUNK, scale_row, 0)
            pltpu.sync_copy(rows_v, out_hbm.at[pl.ds(out0 + g * CHUNK, CHUNK)])
            return carry

        lax.fori_loop(0, n_chunks, chunk_body, 0)

    return k


def kernel(x, table):
    s0, s1 = x.shape
    B = s0 * s1
    D = table.shape[1]
    xf = x.reshape(B // SUB, SUB).astype(jnp.int32)
    out = _build(B, D)(xf, table)
    return out.reshape(s0, s1, D)


# speedup vs baseline: 1.0139x; 1.0139x over previous
import functools
import math

import jax
import jax.numpy as jnp
from jax import lax
from jax.experimental import pallas as pl
from jax.experimental.pallas import tpu as pltpu
from jax.experimental.pallas import tpu_sc as plsc

NC = 2
NS = 16
NW = NC * NS

CHUNK_XR = 16


@functools.lru_cache(maxsize=None)
def _build(R, K, D):
    assert R % (NW * CHUNK_XR) == 0, (R, NW * CHUNK_XR)
    xr_per_w = R // NW
    n_chunks = xr_per_w // CHUNK_XR
    scale = math.sqrt(D)

    mesh = plsc.VectorSubcoreMesh(core_axis_name="c", subcore_axis_name="s")

    @functools.partial(
        pl.kernel,
        mesh=mesh,
        compiler_params=pltpu.CompilerParams(use_tc_tiling_on_sc=False),
        out_type=jax.ShapeDtypeStruct((R, K, D), jnp.float32),
        scratch_types=[
            pltpu.VMEM((CHUNK_XR, K), jnp.int32),
            pltpu.VMEM((CHUNK_XR, K, D), jnp.float32),
            pltpu.SemaphoreType.DMA,
        ],
    )
    def k(x_hbm, table_hbm, out_hbm, idx_v, rows_v, gsem):
        wid = lax.axis_index("s") * NC + lax.axis_index("c")
        xr0 = wid * xr_per_w

        def chunk_body(g, carry):
            xr = xr0 + g * CHUNK_XR
            pltpu.sync_copy(x_hbm.at[pl.ds(xr, CHUNK_XR)], idx_v)
            cps = [
                pltpu.async_copy(table_hbm.at[idx_v.at[j]], rows_v.at[j], gsem)
                for j in range(CHUNK_XR)
            ]
            for cp in cps:
                cp.wait()

            def scale_row(r, c):
                j, r_in = c
                for l in range(D // 16):
                    sl = pl.ds(l * 16, 16)
                    rows_v[j, r_in, sl] = rows_v[j, r_in, sl] * scale
                r_in1 = r_in + 1
                wrap = r_in1 == K
                return (jnp.where(wrap, j + 1, j), jnp.where(wrap, 0, r_in1))

            lax.fori_loop(0, CHUNK_XR * K, scale_row,
                          (jnp.int32(0), jnp.int32(0)))
            pltpu.sync_copy(rows_v, out_hbm.at[pl.ds(xr, CHUNK_XR)])
            return carry

        lax.fori_loop(0, n_chunks, chunk_body, 0)

    return k


def kernel(x, table):
    R, K = x.shape
    D = table.shape[1]
    return _build(R, K, D)(x.astype(jnp.int32), table)


# speedup vs baseline: 1.2826x; 1.2650x over previous
import functools

import jax
import jax.numpy as jnp
from jax import lax
from jax.experimental import pallas as pl
from jax.experimental.pallas import tpu as pltpu
from jax.experimental.pallas import tpu_sc as plsc

NC = 2
NS = 16
NW = NC * NS

IB = 128
SCALE = 8.0


@functools.lru_cache(maxsize=None)
def _build(R, K, D):
    assert D == 64 and R % (NW * IB) == 0, (R, K, D)
    nb = R // (NW * IB)
    n_pairs = K * nb
    assert n_pairs % 2 == 0

    mesh = plsc.VectorSubcoreMesh(core_axis_name="c", subcore_axis_name="s")

    @functools.partial(
        pl.kernel,
        mesh=mesh,
        compiler_params=pltpu.CompilerParams(use_tc_tiling_on_sc=False,
                                             needs_layout_passes=False),
        out_type=jax.ShapeDtypeStruct((K, D // 8, R // IB, 8, IB), jnp.float32),
        scratch_types=[
            pltpu.VMEM((K, nb * IB), jnp.int32),
            pltpu.VMEM((2 * IB, D), jnp.float32),
            pltpu.VMEM((2 * D, IB), jnp.float32),
            pltpu.SemaphoreType.DMA,
            pltpu.SemaphoreType.DMA,
            pltpu.SemaphoreType.DMA,
            pltpu.SemaphoreType.DMA,
            pltpu.SemaphoreType.DMA,
        ],
    )
    def k(xT_hbm, table_hbm, out_hbm, idx_v, g_v, c_v, isem, gsem0, gsem1,
          osem0, osem1):
        wid = lax.axis_index("s") * NC + lax.axis_index("c")
        col0 = wid * (nb * IB)

        icps = [
            pltpu.async_copy(xT_hbm.at[j, pl.ds(col0, nb * IB)],
                             idx_v.at[j], isem)
            for j in range(K)
        ]
        for cp in icps:
            cp.wait()

        gsems = (gsem0, gsem1)
        osems = (osem0, osem1)
        iota = lax.iota(jnp.int32, 16)
        rots = [(iota + r) & 15 for r in range(16)]

        def fire_gather(p, buf):
            j = p // nb
            b = p % nb
            pltpu.async_copy(
                table_hbm.at[idx_v.at[j, pl.ds(b * IB, IB)]],
                g_v.at[pl.ds(buf * IB, IB)], gsems[buf])

        def wait_gather(buf):
            pltpu.make_async_copy(table_hbm.at[pl.ds(0, IB)],
                                  g_v.at[pl.ds(buf * IB, IB)],
                                  gsems[buf]).wait()

        def drain_out(buf):
            pltpu.make_async_copy(table_hbm.at[pl.ds(0, IB)],
                                  g_v.at[pl.ds(buf * IB, IB)],
                                  osems[buf]).wait()

        def process(p, buf):
            wait_gather(buf)

            def block(t, carry):
                bi = t // (D // 16)
                bk = t % (D // 16)
                i2 = bi * 16 + iota
                for r in range(16):
                    kv = bk * 16 + rots[r]
                    v = plsc.load_gather(g_v, [i2 + buf * IB, kv])
                    plsc.store_scatter(c_v, [kv + buf * D, i2], v * SCALE)
                return carry

            lax.fori_loop(0, (IB // 16) * (D // 16), block, 0)

            j = p // nb
            ig = wid * nb + (p % nb)
            for k1 in range(D // 8):
                pltpu.async_copy(c_v.at[pl.ds(buf * D + k1 * 8, 8)],
                                 out_hbm.at[j, k1, ig], osems[buf])

        fire_gather(jnp.int32(0), 0)

        def pair2_body(q, carry):
            p0 = 2 * q
            fire_gather(p0 + 1, 1)

            @pl.when(q > 0)
            def _():
                drain_out(0)

            process(p0, 0)

            @pl.when(q + 1 < n_pairs // 2)
            def _():
                fire_gather(p0 + 2, 0)

            @pl.when(q > 0)
            def _():
                drain_out(1)

            process(p0 + 1, 1)
            return carry

        lax.fori_loop(0, n_pairs // 2, pair2_body, 0)
        drain_out(0)
        drain_out(1)

    return k


def kernel(x, table):
    R, K = x.shape
    D = table.shape[1]
    xT = jnp.swapaxes(x, 0, 1).astype(jnp.int32)
    out5 = _build(R, K, D)(xT, table)
    return out5.transpose(2, 4, 0, 1, 3).reshape(R, K, D)


# speedup vs baseline: 1.7568x; 1.3697x over previous
import functools

import jax
import jax.numpy as jnp
from jax import lax
from jax.experimental import pallas as pl
from jax.experimental.pallas import tpu as pltpu
from jax.experimental.pallas import tpu_sc as plsc

NC = 2
NS = 16
NW = NC * NS

IB = 128
SCALE = 8.0


@functools.lru_cache(maxsize=None)
def _build(R, K, D):
    assert D == 64 and R % (NW * IB) == 0, (R, K, D)
    nb = R // (NW * IB)
    n_pairs = K * nb
    assert n_pairs % 2 == 0

    mesh = plsc.VectorSubcoreMesh(core_axis_name="c", subcore_axis_name="s")

    @functools.partial(
        pl.kernel,
        mesh=mesh,
        compiler_params=pltpu.CompilerParams(use_tc_tiling_on_sc=False,
                                             needs_layout_passes=False),
        out_type=jax.ShapeDtypeStruct((K, D // 8, R // IB, 8, IB), jnp.float32),
        scratch_types=[
            pltpu.VMEM((K, nb * IB), jnp.int32),
            pltpu.VMEM((2 * IB, D), jnp.float32),
            pltpu.VMEM((2 * D, IB), jnp.float32),
            pltpu.SemaphoreType.DMA,
            pltpu.SemaphoreType.DMA,
            pltpu.SemaphoreType.DMA,
            pltpu.SemaphoreType.DMA,
            pltpu.SemaphoreType.DMA,
        ],
    )
    def k(xT_hbm, table_hbm, out_hbm, idx_v, g_v, c_v, isem, gsem0, gsem1,
          osem0, osem1):
        wid = lax.axis_index("s") * NC + lax.axis_index("c")
        col0 = wid * (nb * IB)

        icps = [
            pltpu.async_copy(xT_hbm.at[j, pl.ds(col0, nb * IB)],
                             idx_v.at[j], isem)
            for j in range(K)
        ]
        for cp in icps:
            cp.wait()

        gsems = (gsem0, gsem1)
        osems = (osem0, osem1)
        iota = lax.iota(jnp.int32, 16)
        rots = [(iota + r) & 15 for r in range(16)]

        def fire_gather(p, buf):
            j = p // nb
            b = p % nb
            pltpu.async_copy(
                table_hbm.at[idx_v.at[j, pl.ds(b * IB, IB)]],
                g_v.at[pl.ds(buf * IB, IB)], gsems[buf])

        def wait_gather(buf):
            pltpu.make_async_copy(table_hbm.at[pl.ds(0, IB)],
                                  g_v.at[pl.ds(buf * IB, IB)],
                                  gsems[buf]).wait()

        def drain_out(buf):
            pltpu.make_async_copy(table_hbm.at[pl.ds(0, IB)],
                                  g_v.at[pl.ds(buf * IB, IB)],
                                  osems[buf]).wait()

        def process(p, buf):
            wait_gather(buf)

            g_ref = g_v.at[pl.ds(buf * IB, IB)]
            c_ref = c_v.at[pl.ds(buf * D, D)]

            def block(t, carry):
                bi = t // (D // 16)
                bk = t % (D // 16)
                i2 = bi * 16 + iota
                kvs = [bk * 16 + rots[r] for r in range(16)]
                vals = [plsc.load_gather(g_ref, [i2, kvs[r]]) * SCALE
                        for r in range(16)]
                for r in range(16):
                    plsc.store_scatter(c_ref, [kvs[r], i2], vals[r])
                return carry

            lax.fori_loop(0, (IB // 16) * (D // 16), block, 0)

            j = p // nb
            ig = wid * nb + (p % nb)
            for k1 in range(D // 8):
                pltpu.async_copy(c_v.at[pl.ds(buf * D + k1 * 8, 8)],
                                 out_hbm.at[j, k1, ig], osems[buf])

        fire_gather(jnp.int32(0), 0)

        def pair2_body(q, carry):
            p0 = 2 * q
            fire_gather(p0 + 1, 1)

            @pl.when(q > 0)
            def _():
                drain_out(0)

            process(p0, 0)

            @pl.when(q + 1 < n_pairs // 2)
            def _():
                fire_gather(p0 + 2, 0)

            @pl.when(q > 0)
            def _():
                drain_out(1)

            process(p0 + 1, 1)
            return carry

        lax.fori_loop(0, n_pairs // 2, pair2_body, 0)
        drain_out(0)
        drain_out(1)

    return k


def kernel(x, table):
    R, K = x.shape
    D = table.shape[1]
    xT = jnp.swapaxes(x, 0, 1).astype(jnp.int32)
    out5 = _build(R, K, D)(xT, table)
    return out5.transpose(2, 4, 0, 1, 3).reshape(R, K, D)


# speedup vs baseline: 1.8484x; 1.0522x over previous
import functools

import jax
import jax.numpy as jnp
from jax import lax
from jax.experimental import pallas as pl
from jax.experimental.pallas import tpu as pltpu
from jax.experimental.pallas import tpu_sc as plsc

NC = 2
NS = 16
NW = NC * NS

IB = 128
SCALE = 8.0


@functools.lru_cache(maxsize=None)
def _build(R, K, D):
    assert D == 64 and R % (NW * IB) == 0, (R, K, D)
    nb = R // (NW * IB)
    n_pairs = K * nb
    NBUF = 4
    assert n_pairs % NBUF == 0

    mesh = plsc.VectorSubcoreMesh(core_axis_name="c", subcore_axis_name="s")

    @functools.partial(
        pl.kernel,
        mesh=mesh,
        compiler_params=pltpu.CompilerParams(use_tc_tiling_on_sc=False,
                                             needs_layout_passes=False),
        out_type=jax.ShapeDtypeStruct((K, D // 8, R // IB, 8, IB), jnp.float32),
        scratch_types=[
            pltpu.VMEM((K, nb * IB), jnp.int32),
            pltpu.VMEM((NBUF * IB, D), jnp.float32),
            pltpu.VMEM((NBUF * D, IB), jnp.float32),
            pltpu.SemaphoreType.DMA,
            pltpu.SemaphoreType.DMA,
            pltpu.SemaphoreType.DMA,
            pltpu.SemaphoreType.DMA,
            pltpu.SemaphoreType.DMA,
            pltpu.SemaphoreType.DMA,
            pltpu.SemaphoreType.DMA,
            pltpu.SemaphoreType.DMA,
            pltpu.SemaphoreType.DMA,
        ],
    )
    def k(xT_hbm, table_hbm, out_hbm, idx_v, g_v, c_v, isem, gsem0, gsem1,
          gsem2, gsem3, osem0, osem1, osem2, osem3):
        wid = lax.axis_index("s") * NC + lax.axis_index("c")
        col0 = wid * (nb * IB)

        icps = [
            pltpu.async_copy(xT_hbm.at[j, pl.ds(col0, nb * IB)],
                             idx_v.at[j], isem)
            for j in range(K)
        ]
        for cp in icps:
            cp.wait()

        gsems = (gsem0, gsem1, gsem2, gsem3)
        osems = (osem0, osem1, osem2, osem3)
        iota = lax.iota(jnp.int32, 16)
        rots = [(iota + r) & 15 for r in range(16)]

        def fire_gather(p, buf):
            j = p // nb
            b = p % nb
            pltpu.async_copy(
                table_hbm.at[idx_v.at[j, pl.ds(b * IB, IB)]],
                g_v.at[pl.ds(buf * IB, IB)], gsems[buf])

        def wait_gather(buf):
            pltpu.make_async_copy(table_hbm.at[pl.ds(0, IB)],
                                  g_v.at[pl.ds(buf * IB, IB)],
                                  gsems[buf]).wait()

        def drain_out(buf):
            pltpu.make_async_copy(table_hbm.at[pl.ds(0, IB)],
                                  g_v.at[pl.ds(buf * IB, IB)],
                                  osems[buf]).wait()

        def process(p, buf):
            wait_gather(buf)

            g_ref = g_v.at[pl.ds(buf * IB, IB)]
            c_ref = c_v.at[pl.ds(buf * D, D)]

            def block(t, carry):
                bi = t // (D // 16)
                bk = t % (D // 16)
                i2 = bi * 16 + iota
                kvs = [bk * 16 + rots[r] for r in range(16)]
                vals = [plsc.load_gather(g_ref, [i2, kvs[r]]) * SCALE
                        for r in range(16)]
                for r in range(16):
                    plsc.store_scatter(c_ref, [kvs[r], i2], vals[r])
                return carry

            lax.fori_loop(0, (IB // 16) * (D // 16), block, 0)

            j = p // nb
            ig = wid * nb + (p % nb)
            for k1 in range(D // 8):
                pltpu.async_copy(c_v.at[pl.ds(buf * D + k1 * 8, 8)],
                                 out_hbm.at[j, k1, ig], osems[buf])

        for b in range(NBUF - 1):
            fire_gather(jnp.int32(b), b)

        def pair4_body(q, carry):
            p0 = NBUF * q
            for u in range(NBUF):
                p = p0 + u
                nxt = (u + NBUF - 1) % NBUF

                @pl.when(p + NBUF - 1 < n_pairs)
                def _():
                    fire_gather(p + NBUF - 1, nxt)

                @pl.when(q > 0)
                def _():
                    drain_out(u)

                process(p, u)
            return carry

        lax.fori_loop(0, n_pairs // NBUF, pair4_body, 0)
        for b in range(NBUF):
            drain_out(b)

    return k


def kernel(x, table):
    R, K = x.shape
    D = table.shape[1]
    xT = jnp.swapaxes(x, 0, 1).astype(jnp.int32)
    out5 = _build(R, K, D)(xT, table)
    return out5.transpose(2, 4, 0, 1, 3).reshape(R, K, D)
